# Initial kernel scaffold; baseline (speedup 1.0000x reference)
import jax
import jax.numpy as jnp
from jax.experimental import pallas as pl


def kernel(user, item, user_table, item_table):
    raise NotImplementedError("write your pallas kernel here")


# speedup vs baseline: 1.0424x; 1.0424x over previous
import functools

import jax
import jax.numpy as jnp
from jax import lax
from jax.experimental import pallas as pl
from jax.experimental.pallas import tpu as pltpu
from jax.experimental.pallas import tpu_sc as plsc

B = 16384
D = 128
L = 16
NC = 2
NS = 16
NW = NC * NS
BPW = B // NW
C = 128
NCHUNK = BPW // C


def _mf_body(user_hbm, item_hbm, ut_hbm, it_hbm, out_hbm,
             uidx_v, iidx_v, urows_v, irows_v, prod_v, outbuf_v, sem):
    wid = lax.axis_index("s") * NC + lax.axis_index("c")
    base = wid * BPW

    for c in range(NCHUNK):
        pltpu.sync_copy(user_hbm.at[pl.ds(base + c * C, C)], uidx_v.at[c])
        pltpu.sync_copy(item_hbm.at[pl.ds(base + c * C, C)], iidx_v.at[c])

    for c in range(NCHUNK):
        cu = pltpu.async_copy(ut_hbm.at[uidx_v.at[c]], urows_v, sem)
        ci = pltpu.async_copy(it_hbm.at[iidx_v.at[c]], irows_v, sem)
        cu.wait()
        ci.wait()

        def group(g, _, c=c):
            for j in range(L):
                r = g * L + j
                acc = urows_v[r, pl.ds(0, L)] * irows_v[r, pl.ds(0, L)]
                for k in range(1, D // L):
                    acc = acc + urows_v[r, pl.ds(k * L, L)] * irows_v[r, pl.ds(k * L, L)]
                prod_v[j, pl.ds(0, L)] = acc
            rowidx = lax.broadcasted_iota(jnp.int32, (L,), 0)
            vec = plsc.load_gather(prod_v, [rowidx, jnp.zeros((L,), jnp.int32)])
            for l in range(1, L):
                vec = vec + plsc.load_gather(
                    prod_v, [rowidx, jnp.full((L,), l, jnp.int32)])
            outbuf_v[pl.ds(c * C + g * L, L)] = vec
            return 0

        lax.fori_loop(0, C // L, group, 0)

    def sig(t, _):
        x = outbuf_v[pl.ds(t * L, L)]
        outbuf_v[pl.ds(t * L, L)] = 1.0 / (1.0 + jnp.exp(-x))
        return 0

    lax.fori_loop(0, BPW // L, sig, 0)
    pltpu.sync_copy(outbuf_v, out_hbm.at[pl.ds(base, BPW)])


@jax.jit
def kernel(user, item, user_table, item_table):
    mesh = plsc.VectorSubcoreMesh(
        core_axis_name="c", subcore_axis_name="s",
        num_cores=NC, num_subcores=NS)
    run = pl.kernel(
        _mf_body,
        out_type=jax.ShapeDtypeStruct((B,), jnp.float32),
        mesh=mesh,
        scratch_types=[
            pltpu.VMEM((NCHUNK, C), jnp.int32),
            pltpu.VMEM((NCHUNK, C), jnp.int32),
            pltpu.VMEM((C, D), jnp.float32),
            pltpu.VMEM((C, D), jnp.float32),
            pltpu.VMEM((L, L), jnp.float32),
            pltpu.VMEM((BPW,), jnp.float32),
            pltpu.SemaphoreType.DMA,
        ],
        compiler_params=pltpu.CompilerParams(needs_layout_passes=False),
    )
    return run(user.astype(jnp.int32), item.astype(jnp.int32),
               user_table, item_table)


# speedup vs baseline: 1.1679x; 1.1203x over previous
import functools

import jax
import jax.numpy as jnp
from jax import lax
from jax.experimental import pallas as pl
from jax.experimental.pallas import tpu as pltpu
from jax.experimental.pallas import tpu_sc as plsc

B = 16384
D = 128
L = 16
NC = 2
NS = 16
NW = NC * NS
BPW = B // NW
C = 128
NCHUNK = BPW // C
PP = L + 1


def _mf_body(user_hbm, item_hbm, ut_hbm, it_hbm, out_hbm,
             uidx_v, iidx_v, urows0, irows0, urows1, irows1,
             prod_v, outbuf_v, sem0, sem1):
    wid = lax.axis_index("s") * NC + lax.axis_index("c")
    base = wid * BPW

    for c in range(NCHUNK):
        pltpu.sync_copy(user_hbm.at[pl.ds(base + c * C, C)], uidx_v.at[c])
        pltpu.sync_copy(item_hbm.at[pl.ds(base + c * C, C)], iidx_v.at[c])

    bufs = ((urows0, irows0, sem0), (urows1, irows1, sem1))
    inflight = [None, None]

    def fire(c):
        ub, ib, sem = bufs[c % 2]
        cu = pltpu.async_copy(ut_hbm.at[uidx_v.at[c]], ub, sem)
        ci = pltpu.async_copy(it_hbm.at[iidx_v.at[c]], ib, sem)
        inflight[c % 2] = (cu, ci)

    fire(0)
    for c in range(NCHUNK):
        if c + 1 < NCHUNK:
            fire(c + 1)
        cu, ci = inflight[c % 2]
        cu.wait()
        ci.wait()
        ub, ib, _ = bufs[c % 2]

        def group(g, _, c=c, ub=ub, ib=ib):
            for j in range(L):
                r = g * L + j
                acc = ub[r, pl.ds(0, L)] * ib[r, pl.ds(0, L)]
                for k in range(1, D // L):
                    acc = acc + ub[r, pl.ds(k * L, L)] * ib[r, pl.ds(k * L, L)]
                prod_v[j, pl.ds(0, L)] = acc
            rowidx = lax.broadcasted_iota(jnp.int32, (L,), 0)
            vec = plsc.load_gather(prod_v, [rowidx, jnp.zeros((L,), jnp.int32)])
            for l in range(1, L):
                vec = vec + plsc.load_gather(
                    prod_v, [rowidx, jnp.full((L,), l, jnp.int32)])
            vec = 1.0 / (1.0 + jnp.exp(-vec))
            outbuf_v[pl.ds(c * C + g * L, L)] = vec
            return 0

        lax.fori_loop(0, C // L, group, 0)

    pltpu.sync_copy(outbuf_v, out_hbm.at[pl.ds(base, BPW)])


@jax.jit
def kernel(user, item, user_table, item_table):
    mesh = plsc.VectorSubcoreMesh(
        core_axis_name="c", subcore_axis_name="s",
        num_cores=NC, num_subcores=NS)
    run = pl.kernel(
        _mf_body,
        out_type=jax.ShapeDtypeStruct((B,), jnp.float32),
        mesh=mesh,
        scratch_types=[
            pltpu.VMEM((NCHUNK, C), jnp.int32),
            pltpu.VMEM((NCHUNK, C), jnp.int32),
            pltpu.VMEM((C, D), jnp.float32),
            pltpu.VMEM((C, D), jnp.float32),
            pltpu.VMEM((C, D), jnp.float32),
            pltpu.VMEM((C, D), jnp.float32),
            pltpu.VMEM((L, PP), jnp.float32),
            pltpu.VMEM((BPW,), jnp.float32),
            pltpu.SemaphoreType.DMA,
            pltpu.SemaphoreType.DMA,
        ],
        compiler_params=pltpu.CompilerParams(needs_layout_passes=False),
    )
    return run(user.astype(jnp.int32), item.astype(jnp.int32),
               user_table, item_table)


# speedup vs baseline: 1.2272x; 1.0508x over previous
import functools

import jax
import jax.numpy as jnp
from jax import lax
from jax.experimental import pallas as pl
from jax.experimental.pallas import tpu as pltpu
from jax.experimental.pallas import tpu_sc as plsc

B = 16384
D = 128
L = 16
NC = 2
NS = 16
NW = NC * NS
BPW = B // NW
C = 128
NCHUNK = BPW // C
NBUF = 3
PP = L + 1


def _mf_body(user_hbm, item_hbm, ut_hbm, it_hbm, out_hbm,
             uidx_v, iidx_v, urows, irows, prod_v, outbuf_v, sems, isem):
    wid = lax.axis_index("s") * NC + lax.axis_index("c")
    base = wid * BPW

    ciu = pltpu.async_copy(user_hbm.at[pl.ds(base, BPW)], uidx_v, isem)
    cii = pltpu.async_copy(item_hbm.at[pl.ds(base, BPW)], iidx_v, isem)
    ciu.wait()
    cii.wait()

    inflight = [None] * NBUF

    def fire(c):
        s = c % NBUF
        cu = pltpu.async_copy(
            ut_hbm.at[uidx_v.at[pl.ds(c * C, C)]], urows.at[s], sems.at[s])
        ci = pltpu.async_copy(
            it_hbm.at[iidx_v.at[pl.ds(c * C, C)]], irows.at[s], sems.at[s])
        inflight[s] = (cu, ci)

    for c in range(min(NBUF - 1, NCHUNK)):
        fire(c)

    for c in range(NCHUNK):
        if c + NBUF - 1 < NCHUNK:
            fire(c + NBUF - 1)
        s = c % NBUF
        cu, ci = inflight[s]
        cu.wait()
        ci.wait()

        def group(g, _, c=c, s=s):
            for j in range(L):
                r = g * L + j
                acc = urows[s, r, pl.ds(0, L)] * irows[s, r, pl.ds(0, L)]
                for k in range(1, D // L):
                    acc = acc + (urows[s, r, pl.ds(k * L, L)]
                                 * irows[s, r, pl.ds(k * L, L)])
                prod_v[j, pl.ds(0, L)] = acc
            rowidx = lax.broadcasted_iota(jnp.int32, (L,), 0)
            vec = plsc.load_gather(prod_v, [rowidx, jnp.zeros((L,), jnp.int32)])
            for l in range(1, L):
                vec = vec + plsc.load_gather(
                    prod_v, [rowidx, jnp.full((L,), l, jnp.int32)])
            vec = 1.0 / (1.0 + jnp.exp(-vec))
            outbuf_v[pl.ds(c * C + g * L, L)] = vec
            return 0

        lax.fori_loop(0, C // L, group, 0)

    pltpu.sync_copy(outbuf_v, out_hbm.at[pl.ds(base, BPW)])


@jax.jit
def kernel(user, item, user_table, item_table):
    mesh = plsc.VectorSubcoreMesh(
        core_axis_name="c", subcore_axis_name="s",
        num_cores=NC, num_subcores=NS)
    run = pl.kernel(
        _mf_body,
        out_type=jax.ShapeDtypeStruct((B,), jnp.float32),
        mesh=mesh,
        scratch_types=[
            pltpu.VMEM((BPW,), jnp.int32),
            pltpu.VMEM((BPW,), jnp.int32),
            pltpu.VMEM((NBUF, C, D), jnp.float32),
            pltpu.VMEM((NBUF, C, D), jnp.float32),
            pltpu.VMEM((L, PP), jnp.float32),
            pltpu.VMEM((BPW,), jnp.float32),
            pltpu.SemaphoreType.DMA((NBUF,)),
            pltpu.SemaphoreType.DMA,
        ],
        compiler_params=pltpu.CompilerParams(needs_layout_passes=False),
    )
    return run(user.astype(jnp.int32), item.astype(jnp.int32),
               user_table, item_table)
